# baseline (speedup 1.0000x reference)
import jax
import jax.numpy as jnp
from jax.experimental import pallas as pl
from jax.experimental.pallas import tpu as pltpu

N = 15
NP = 16
D = 512
H = 3
L = 3
NEG = -1e30


def _layer_kernel(x0_ref, news_ref, ws0_ref, ws1_ref, ws2_ref, wd1_ref,
                  wd2_ref, asT_ref, adT_ref, b_ref, out_ref):
    i = pl.program_id(0)

    x = jnp.where(i == 0, x0_ref[...], out_ref[...])

    ws0 = ws0_ref[0, 0]
    w_avg = (ws0[:, :D] + ws0[:, D:2 * D] + ws0[:, 2 * D:]) * (1.0 / 3.0)
    acc = jnp.dot(news_ref[0], w_avg, preferred_element_type=jnp.float32)

    src_ok = jax.lax.broadcasted_iota(jnp.int32, (NP, H * NP), 0) < N

    for r in (1, 2):
        ws_r = ws1_ref[0, 0] if r == 1 else ws2_ref[0, 0]
        wd_r = wd1_ref[0, 0] if r == 1 else wd2_ref[0, 0]
        a_sT = asT_ref[0, r]
        a_dT = adT_ref[0, r]
        hs = jnp.dot(x, ws_r, preferred_element_type=jnp.float32)
        hd = jnp.dot(x, wd_r, preferred_element_type=jnp.float32)

        blocks = []
        for h in range(H):
            al_s = jnp.dot(hs[:, h * D:(h + 1) * D], a_sT[:, h:h + 1],
                           preferred_element_type=jnp.float32)
            al_d = jnp.dot(hd[:, h * D:(h + 1) * D], a_dT[:, h:h + 1],
                           preferred_element_type=jnp.float32)
            blocks.append(al_s + jnp.transpose(al_d))
        alpha = jnp.concatenate(blocks, axis=1)
        alpha = jnp.where(alpha > 0, alpha, 0.2 * alpha)
        alpha = jnp.where(src_ok, alpha, NEG)
        amax = jnp.max(alpha, axis=0, keepdims=True)
        e = jnp.exp(alpha - amax)
        denom = jnp.sum(e, axis=0, keepdims=True)
        att = e / (denom + 1e-16)

        res = jax.lax.dot_general(att, hs, (((0,), (0,)), ((), ())),
                                  preferred_element_type=jnp.float32)
        acc = acc + (1.0 / H) * (res[:NP, :D]
                                 + res[NP:2 * NP, D:2 * D]
                                 + res[2 * NP:, 2 * D:])

    b = b_ref[0]
    acc = acc + (b[0] + b[1] + b[2])[None, :]
    out_ref[...] = jax.nn.sigmoid(acc * (1.0 / 3.0))


@jax.jit
def kernel(company_features, daily_news_features, W_src, W_dst, att_src,
           att_dst, bias):
    x0 = jnp.zeros((NP, D), jnp.float32).at[:N].set(company_features)
    news = jnp.zeros((L, NP, D), jnp.float32).at[:, :N].set(
        daily_news_features)
    a_sT = att_src.transpose(0, 1, 3, 2)
    a_dT = att_dst.transpose(0, 1, 3, 2)

    def wspec(r):
        return pl.BlockSpec((1, 1, D, H * D), lambda i, _r=r: (i, _r, 0, 0))

    out = pl.pallas_call(
        _layer_kernel,
        grid=(L,),
        in_specs=[
            pl.BlockSpec((NP, D), lambda i: (0, 0)),
            pl.BlockSpec((1, NP, D), lambda i: (i, 0, 0)),
            wspec(0), wspec(1), wspec(2),
            wspec(1), wspec(2),
            pl.BlockSpec((1, 3, D, H), lambda i: (i, 0, 0, 0)),
            pl.BlockSpec((1, 3, D, H), lambda i: (i, 0, 0, 0)),
            pl.BlockSpec((1, 3, D), lambda i: (i, 0, 0)),
        ],
        out_specs=pl.BlockSpec((NP, D), lambda i: (0, 0)),
        out_shape=jax.ShapeDtypeStruct((NP, D), jnp.float32),
        compiler_params=pltpu.CompilerParams(
            dimension_semantics=("arbitrary",)),
    )(x0, news, W_src, W_src, W_src, W_dst, W_dst, a_sT, a_dT, bias)
    return out[:N]


# speedup vs baseline: 1.3071x; 1.3071x over previous
import jax
import jax.numpy as jnp
from jax.experimental import pallas as pl
from jax.experimental.pallas import tpu as pltpu

N = 15
NP = 16
D = 512
H = 3
L = 3
NEG = -1e30


def _layer_kernel(x0_ref, news_ref, ws0_ref, ws1_ref, ws2_ref, wd1_ref,
                  wd2_ref, as_ref, ad_ref, b_ref, out_ref):
    i = pl.program_id(0)

    x = jnp.where(i == 0, x0_ref[...], out_ref[...])

    ws0 = ws0_ref[0, 0]
    w_avg = (ws0[:, :D] + ws0[:, D:2 * D] + ws0[:, 2 * D:]) * (1.0 / 3.0)
    acc = jnp.dot(news_ref[0], w_avg, preferred_element_type=jnp.float32)

    src_ok = jax.lax.broadcasted_iota(jnp.int32, (NP, H * NP), 0) < N

    for r in (1, 2):
        ws_r = ws1_ref[0, 0] if r == 1 else ws2_ref[0, 0]
        wd_r = wd1_ref[0, 0] if r == 1 else wd2_ref[0, 0]
        a_s = as_ref[0, r]
        a_d = ad_ref[0, r]
        hs = jnp.dot(x, ws_r, preferred_element_type=jnp.float32)
        hd = jnp.dot(x, wd_r, preferred_element_type=jnp.float32)

        blocks = []
        for h in range(H):
            hs_h = hs[:, h * D:(h + 1) * D]
            hd_h = hd[:, h * D:(h + 1) * D]
            al_s = jnp.sum(hs_h * a_s[h][None, :], axis=1, keepdims=True)
            al_d = jnp.sum(hd_h * a_d[h][None, :], axis=1, keepdims=True)
            blocks.append(al_s + jnp.transpose(al_d))
        alpha = jnp.concatenate(blocks, axis=1)
        alpha = jnp.where(alpha > 0, alpha, 0.2 * alpha)
        alpha = jnp.where(src_ok, alpha, NEG)
        amax = jnp.max(alpha, axis=0, keepdims=True)
        e = jnp.exp(alpha - amax)
        denom = jnp.sum(e, axis=0, keepdims=True)
        att = e / (denom + 1e-16)

        res = jax.lax.dot_general(att, hs, (((0,), (0,)), ((), ())),
                                  preferred_element_type=jnp.float32)
        acc = acc + (1.0 / H) * (res[:NP, :D]
                                 + res[NP:2 * NP, D:2 * D]
                                 + res[2 * NP:, 2 * D:])

    b = b_ref[0]
    acc = acc + (b[0] + b[1] + b[2])[None, :]
    out_ref[...] = jax.nn.sigmoid(acc * (1.0 / 3.0))


@jax.jit
def kernel(company_features, daily_news_features, W_src, W_dst, att_src,
           att_dst, bias):
    x0 = jnp.zeros((NP, D), jnp.float32).at[:N].set(company_features)
    news = jnp.zeros((L, NP, D), jnp.float32).at[:, :N].set(
        daily_news_features)

    def wspec(r):
        return pl.BlockSpec((1, 1, D, H * D), lambda i, _r=r: (i, _r, 0, 0))

    out = pl.pallas_call(
        _layer_kernel,
        grid=(L,),
        in_specs=[
            pl.BlockSpec((NP, D), lambda i: (0, 0)),
            pl.BlockSpec((1, NP, D), lambda i: (i, 0, 0)),
            wspec(0), wspec(1), wspec(2),
            wspec(1), wspec(2),
            pl.BlockSpec((1, 3, H, D), lambda i: (i, 0, 0, 0)),
            pl.BlockSpec((1, 3, H, D), lambda i: (i, 0, 0, 0)),
            pl.BlockSpec((1, 3, D), lambda i: (i, 0, 0)),
        ],
        out_specs=pl.BlockSpec((NP, D), lambda i: (0, 0)),
        out_shape=jax.ShapeDtypeStruct((NP, D), jnp.float32),
        compiler_params=pltpu.CompilerParams(
            dimension_semantics=("arbitrary",)),
    )(x0, news, W_src, W_src, W_src, W_dst, W_dst, att_src, att_dst, bias)
    return out[:N]


# speedup vs baseline: 1.3699x; 1.0480x over previous
import jax
import jax.numpy as jnp
from jax.experimental import pallas as pl
from jax.experimental.pallas import tpu as pltpu

N = 15
NP = 16
D = 512
H = 3
L = 3
NEG = -1e30


def _layer_kernel(x0_ref, news_ref, ws0_ref, ws1_ref, ws2_ref, wd1_ref,
                  wd2_ref, as_ref, ad_ref, b_ref, out_ref):
    i = pl.program_id(0)

    x = jnp.where(i == 0, x0_ref[...], out_ref[...])

    ws0 = ws0_ref[0, 0]
    w_avg = (ws0[:, :D] + ws0[:, D:2 * D] + ws0[:, 2 * D:]) * (1.0 / 3.0)
    acc = jnp.dot(news_ref[0], w_avg, preferred_element_type=jnp.float32)

    src_ok = jax.lax.broadcasted_iota(jnp.int32, (NP, NP), 0) < N

    for r in (1, 2):
        ws_r = ws1_ref[0, 0] if r == 1 else ws2_ref[0, 0]
        wd_r = wd1_ref[0, 0] if r == 1 else wd2_ref[0, 0]
        a_s = as_ref[0, r]
        a_d = ad_ref[0, r]
        hs = jnp.dot(x, ws_r, preferred_element_type=jnp.float32)
        hd = jnp.dot(x, wd_r, preferred_element_type=jnp.float32)
        for h in range(H):
            hs_h = hs[:, h * D:(h + 1) * D]
            hd_h = hd[:, h * D:(h + 1) * D]
            al_s = jnp.sum(hs_h * a_s[h][None, :], axis=1, keepdims=True)
            al_d = jnp.sum(hd_h * a_d[h][None, :], axis=1, keepdims=True)
            alpha = al_s + jnp.transpose(al_d)
            alpha = jnp.where(alpha > 0, alpha, 0.2 * alpha)
            alpha = jnp.where(src_ok, alpha, NEG)
            amax = jnp.max(alpha, axis=0, keepdims=True)
            e = jnp.exp(alpha - amax)
            denom = jnp.sum(e, axis=0, keepdims=True)
            att = e / (denom + 1e-16)
            acc = acc + (1.0 / H) * jax.lax.dot_general(
                att, hs_h, (((0,), (0,)), ((), ())),
                preferred_element_type=jnp.float32)

    b = b_ref[0]
    acc = acc + (b[0] + b[1] + b[2])[None, :]
    out_ref[...] = jax.nn.sigmoid(acc * (1.0 / 3.0))


@jax.jit
def kernel(company_features, daily_news_features, W_src, W_dst, att_src,
           att_dst, bias):
    x0 = jnp.zeros((NP, D), jnp.float32).at[:N].set(company_features)
    news = jnp.zeros((L, NP, D), jnp.float32).at[:, :N].set(
        daily_news_features)

    def wspec(r):
        return pl.BlockSpec((1, 1, D, H * D), lambda i, _r=r: (i, _r, 0, 0))

    out = pl.pallas_call(
        _layer_kernel,
        grid=(L,),
        in_specs=[
            pl.BlockSpec((NP, D), lambda i: (0, 0)),
            pl.BlockSpec((1, NP, D), lambda i: (i, 0, 0)),
            wspec(0), wspec(1), wspec(2),
            wspec(1), wspec(2),
            pl.BlockSpec((1, 3, H, D), lambda i: (i, 0, 0, 0)),
            pl.BlockSpec((1, 3, H, D), lambda i: (i, 0, 0, 0)),
            pl.BlockSpec((1, 3, D), lambda i: (i, 0, 0)),
        ],
        out_specs=pl.BlockSpec((NP, D), lambda i: (0, 0)),
        out_shape=jax.ShapeDtypeStruct((NP, D), jnp.float32),
        compiler_params=pltpu.CompilerParams(
            dimension_semantics=("arbitrary",)),
    )(x0, news, W_src, W_src, W_src, W_dst, W_dst, att_src, att_dst, bias)
    return out[:N]
